# baseline (speedup 1.0000x reference)
import functools

import jax
import jax.numpy as jnp
from jax import lax
from jax.experimental import pallas as pl
from jax.experimental.pallas import tpu as pltpu
from jax.experimental.pallas import tpu_sc as plsc

N = 10000
E = 320000
D = 128
G = 64

NC = 2
NS = 16
NW = NC * NS
EPW = 10240
E_PAD = NW * EPW
K = 128
CHUNKS = EPW // K
NP = 10240
RPT = NP // NS
ZR = 32
DEGW = 128

_mesh = plsc.VectorSubcoreMesh(
    core_axis_name="c", subcore_axis_name="s", num_cores=NC, num_subcores=NS)


def _fill_const(ref, rows, width, val):
    def body(i, _):
        r = i // (width // 16)
        c = (i % (width // 16)) * 16
        ref[r, pl.ds(c, 16)] = jnp.full((16,), val, jnp.float32)
        return 0
    lax.fori_loop(0, rows * (width // 16), body, 0)


_DEG_KERNEL_ARGS = dict(
    out_type=jax.ShapeDtypeStruct((NC * NP, DEGW), jnp.float32),
    mesh=_mesh,
    scratch_types=[
        pltpu.VMEM((CHUNKS, K), jnp.int32),
        pltpu.VMEM((K, DEGW), jnp.float32),
        pltpu.VMEM((ZR, DEGW), jnp.float32),
        pltpu.VMEM_SHARED((NP, DEGW), jnp.float32),
        pltpu.SemaphoreType.DMA,
    ],
)


def _deg_body(col3_hbm, out_hbm, colb, ones_v, zbuf_v, acc_sh, sem):
    cid = lax.axis_index("c")
    sid = lax.axis_index("s")
    wid = cid * NS + sid

    idx_cp = pltpu.async_copy(col3_hbm.at[wid], colb, sem)
    _fill_const(zbuf_v, ZR, DEGW, 0.0)
    _fill_const(ones_v, K, DEGW, 1.0)
    for z in range(RPT // ZR):
        pltpu.sync_copy(zbuf_v, acc_sh.at[pl.ds(sid * RPT + z * ZR, ZR)])
    idx_cp.wait()
    plsc.subcore_barrier()

    def body(g, _):
        pltpu.sync_copy(ones_v, acc_sh.at[colb.at[g]], add=True)
        return 0
    lax.fori_loop(0, CHUNKS, body, 0)

    plsc.subcore_barrier()
    pltpu.sync_copy(acc_sh.at[pl.ds(sid * RPT, RPT)],
                    out_hbm.at[pl.ds(cid * NP + sid * RPT, RPT)])


_deg_kernel = pl.kernel(_deg_body, **_DEG_KERNEL_ARGS)


_CONV_KERNEL_ARGS = dict(
    out_type=jax.ShapeDtypeStruct((NC * NP, D), jnp.float32),
    mesh=_mesh,
    scratch_types=[
        pltpu.VMEM((K,), jnp.int32),
        pltpu.VMEM((K,), jnp.int32),
        pltpu.VMEM((CHUNKS, K), jnp.int32),
        pltpu.VMEM((K, D), jnp.float32),
        pltpu.VMEM((K, D), jnp.float32),
        pltpu.VMEM((ZR, D), jnp.float32),
        pltpu.VMEM_SHARED((NP, D), jnp.float32),
        pltpu.SemaphoreType.DMA,
        pltpu.SemaphoreType.DMA,
        pltpu.SemaphoreType.DMA,
        pltpu.SemaphoreType.DMA,
    ],
)


def _conv_body(u_hbm, row_hbm, col3_hbm, out_hbm,
               ri0, ri1, colb, r0_v, r1_v, zbuf_v, acc_sh,
               si0, si1, sg0, sg1):
    cid = lax.axis_index("c")
    sid = lax.axis_index("s")
    wid = cid * NS + sid
    ebase = wid * EPW

    col_cp = pltpu.async_copy(col3_hbm.at[wid], colb, sg1)
    _fill_const(zbuf_v, ZR, D, 0.0)
    for z in range(RPT // ZR):
        pltpu.sync_copy(zbuf_v, acc_sh.at[pl.ds(sid * RPT + z * ZR, ZR)])
    col_cp.wait()
    plsc.subcore_barrier()

    def idx_load(g, buf, sem):
        return pltpu.async_copy(row_hbm.at[pl.ds(ebase + g * K, K)], buf, sem)

    def idx_wait(buf, sem):
        pltpu.make_async_copy(row_hbm.at[pl.ds(0, K)], buf, sem).wait()

    def gather(ibuf, buf, sem):
        return pltpu.async_copy(u_hbm.at[ibuf], buf, sem)

    def gwait(buf, sem):
        pltpu.make_async_copy(u_hbm.at[ri0], buf, sem).wait()

    def scatter(g, buf):
        pltpu.sync_copy(buf, acc_sh.at[colb.at[g]], add=True)

    idx_load(0, ri0, si0)
    idx_wait(ri0, si0)
    gather(ri0, r0_v, sg0)
    idx_load(1, ri1, si1)

    def body(i, _):
        a = 2 * i
        idx_wait(ri1, si1)
        gather(ri1, r1_v, sg1)
        gwait(r0_v, sg0)

        @pl.when(a + 2 < CHUNKS)
        def _():
            idx_load(a + 2, ri0, si0)
        scatter(a, r0_v)

        @pl.when(a + 2 < CHUNKS)
        def _():
            idx_wait(ri0, si0)
            gather(ri0, r0_v, sg0)
        gwait(r1_v, sg1)

        @pl.when(a + 3 < CHUNKS)
        def _():
            idx_load(a + 3, ri1, si1)
        scatter(a + 1, r1_v)
        return 0
    lax.fori_loop(0, CHUNKS // 2, body, 0)

    plsc.subcore_barrier()
    pltpu.sync_copy(acc_sh.at[pl.ds(sid * RPT, RPT)],
                    out_hbm.at[pl.ds(cid * NP + sid * RPT, RPT)])


_conv_kernel = pl.kernel(_conv_body, **_CONV_KERNEL_ARGS)


def _tc1a_body(x_ref, fcwt_ref, fcb_ref, w1t_ref, t1_ref):
    h0 = jnp.dot(x_ref[...], fcwt_ref[...],
                 preferred_element_type=jnp.float32) + fcb_ref[...]
    t1_ref[...] = jnp.dot(h0, w1t_ref[...], preferred_element_type=jnp.float32)


def _tc1b_body(t1_ref, degp_ref, u1_ref, dis_ref):
    deg = degp_ref[0, 0:N, 0:1] + degp_ref[1, 0:N, 0:1] + 1.0
    dis = lax.rsqrt(deg)
    dis_ref[...] = dis
    u1_ref[...] = t1_ref[...] * dis


def _tc2_body(acc_ref, u_ref, dis_ref, b_ref, wt_ref, uo_ref):
    s = acc_ref[0, 0:N, :] + acc_ref[1, 0:N, :] + u_ref[...]
    dis = dis_ref[...]
    h = jnp.maximum(s * dis + b_ref[...], 0.0)
    t = jnp.dot(h, wt_ref[...], preferred_element_type=jnp.float32)
    uo_ref[...] = t * dis


def _tc3_body(acc_ref, u_ref, dis_ref, b_ref, batch_ref, out_ref):
    s = acc_ref[0, 0:N, :] + acc_ref[1, 0:N, :] + u_ref[...]
    h = jnp.maximum(s * dis_ref[...] + b_ref[...], 0.0)
    seg = lax.broadcasted_iota(jnp.int32, (G, N), 0)
    onehot = (seg == jnp.broadcast_to(batch_ref[...], (G, N))
              ).astype(jnp.float32)
    sums = jnp.dot(onehot, h, preferred_element_type=jnp.float32)
    counts = jnp.sum(onehot, axis=1, keepdims=True)
    out_ref[...] = sums / jnp.maximum(counts, 1.0)


_tc1a = pl.pallas_call(
    _tc1a_body, out_shape=jax.ShapeDtypeStruct((N, D), jnp.float32))

_tc1b = pl.pallas_call(
    _tc1b_body,
    out_shape=(jax.ShapeDtypeStruct((N, D), jnp.float32),
               jax.ShapeDtypeStruct((N, 1), jnp.float32)))

_tc2 = pl.pallas_call(
    _tc2_body, out_shape=jax.ShapeDtypeStruct((N, D), jnp.float32))

_tc3 = pl.pallas_call(
    _tc3_body, out_shape=jax.ShapeDtypeStruct((G, D), jnp.float32))


def kernel(x, edge_index, batch, fc_W, fc_b, W1, b1, W2, b2):
    pad = E_PAD - E
    row1 = jnp.concatenate([edge_index[0], jnp.zeros((pad,), jnp.int32)])
    col3 = jnp.concatenate(
        [edge_index[1], jnp.full((pad,), NP - 1, jnp.int32)]).reshape(NW, CHUNKS, K)

    degp = _deg_kernel(col3).reshape(NC, NP, DEGW)
    t1 = _tc1a(x, fc_W.T, fc_b.reshape(1, D), W1.T)
    u1, dis = _tc1b(t1, degp)

    acc1 = _conv_kernel(u1, row1, col3).reshape(NC, NP, D)
    u2 = _tc2(acc1, u1, dis, b1.reshape(1, D), W2.T)

    acc2 = _conv_kernel(u2, row1, col3).reshape(NC, NP, D)
    out = _tc3(acc2, u2, dis, b2.reshape(1, D), batch.reshape(1, N))
    return out


# speedup vs baseline: 3.1507x; 3.1507x over previous
import functools

import jax
import jax.numpy as jnp
from jax import lax
from jax.experimental import pallas as pl
from jax.experimental.pallas import tpu as pltpu
from jax.experimental.pallas import tpu_sc as plsc

N = 10000
E = 320000
D = 128
G = 64

NC = 2
NS = 16
NW = NC * NS
EPW = 10240
E_PAD = NW * EPW
K = 128
CHUNKS = EPW // K
NP = 10240
RPT = NP // NS
ZR = 32
DEGW = 128

_mesh = plsc.VectorSubcoreMesh(
    core_axis_name="c", subcore_axis_name="s", num_cores=NC, num_subcores=NS)


def _fill_const(ref, rows, width, val):
    def body(i, _):
        r = i // (width // 16)
        c = (i % (width // 16)) * 16
        ref[r, pl.ds(c, 16)] = jnp.full((16,), val, jnp.float32)
        return 0
    lax.fori_loop(0, rows * (width // 16), body, 0)


_DEG_KERNEL_ARGS = dict(
    out_type=jax.ShapeDtypeStruct((NC * NP, DEGW), jnp.float32),
    mesh=_mesh,
    scratch_types=[
        pltpu.VMEM((CHUNKS, K), jnp.int32),
        pltpu.VMEM((K, DEGW), jnp.float32),
        pltpu.VMEM((ZR, DEGW), jnp.float32),
        pltpu.VMEM_SHARED((NP, DEGW), jnp.float32),
        pltpu.SemaphoreType.DMA,
    ],
)


def _deg_body(col3_hbm, out_hbm, colb, ones_v, zbuf_v, acc_sh, sem):
    cid = lax.axis_index("c")
    sid = lax.axis_index("s")
    wid = cid * NS + sid

    idx_cp = pltpu.async_copy(col3_hbm.at[wid], colb, sem)
    _fill_const(zbuf_v, ZR, DEGW, 0.0)
    _fill_const(ones_v, K, DEGW, 1.0)
    for z in range(RPT // ZR):
        pltpu.sync_copy(zbuf_v, acc_sh.at[pl.ds(sid * RPT + z * ZR, ZR)])
    idx_cp.wait()
    plsc.subcore_barrier()

    def body(g, _):
        pltpu.sync_copy(ones_v, acc_sh.at[colb.at[g]], add=True)
        return 0
    lax.fori_loop(0, CHUNKS, body, 0)

    plsc.subcore_barrier()
    pltpu.sync_copy(acc_sh.at[pl.ds(sid * RPT, RPT)],
                    out_hbm.at[pl.ds(cid * NP + sid * RPT, RPT)])


_deg_kernel = pl.kernel(_deg_body, **_DEG_KERNEL_ARGS)


_CONV_KERNEL_ARGS = dict(
    out_type=jax.ShapeDtypeStruct((NC * NP, D), jnp.float32),
    mesh=_mesh,
    scratch_types=[
        pltpu.VMEM((K,), jnp.int32),
        pltpu.VMEM((K,), jnp.int32),
        pltpu.VMEM((CHUNKS, K), jnp.int32),
        pltpu.VMEM((K, D), jnp.float32),
        pltpu.VMEM((K, D), jnp.float32),
        pltpu.VMEM((ZR, D), jnp.float32),
        pltpu.VMEM_SHARED((NP, D), jnp.float32),
        pltpu.SemaphoreType.DMA,
        pltpu.SemaphoreType.DMA,
        pltpu.SemaphoreType.DMA,
        pltpu.SemaphoreType.DMA,
    ],
)


def _conv_body(u_hbm, row_hbm, col3_hbm, out_hbm,
               ri0, ri1, colb, r0_v, r1_v, zbuf_v, acc_sh,
               si0, si1, sg0, sg1):
    cid = lax.axis_index("c")
    sid = lax.axis_index("s")
    wid = cid * NS + sid
    ebase = wid * EPW

    col_cp = pltpu.async_copy(col3_hbm.at[wid], colb, sg1)
    _fill_const(zbuf_v, ZR, D, 0.0)
    for z in range(RPT // ZR):
        pltpu.sync_copy(zbuf_v, acc_sh.at[pl.ds(sid * RPT + z * ZR, ZR)])
    col_cp.wait()
    plsc.subcore_barrier()

    def idx_load(g, buf, sem):
        return pltpu.async_copy(row_hbm.at[pl.ds(ebase + g * K, K)], buf, sem)

    def idx_wait(buf, sem):
        pltpu.make_async_copy(row_hbm.at[pl.ds(0, K)], buf, sem).wait()

    def gather(ibuf, buf, sem):
        return pltpu.async_copy(u_hbm.at[ibuf], buf, sem)

    def gwait(buf, sem):
        pltpu.make_async_copy(u_hbm.at[ri0], buf, sem).wait()

    def scatter(g, buf):
        pltpu.sync_copy(buf, acc_sh.at[colb.at[g]], add=True)

    idx_load(0, ri0, si0)
    idx_wait(ri0, si0)
    gather(ri0, r0_v, sg0)
    idx_load(1, ri1, si1)

    def body(i, _):
        a = 2 * i
        idx_wait(ri1, si1)
        gather(ri1, r1_v, sg1)
        gwait(r0_v, sg0)

        @pl.when(a + 2 < CHUNKS)
        def _():
            idx_load(a + 2, ri0, si0)
        scatter(a, r0_v)

        @pl.when(a + 2 < CHUNKS)
        def _():
            idx_wait(ri0, si0)
            gather(ri0, r0_v, sg0)
        gwait(r1_v, sg1)

        @pl.when(a + 3 < CHUNKS)
        def _():
            idx_load(a + 3, ri1, si1)
        scatter(a + 1, r1_v)
        return 0
    lax.fori_loop(0, CHUNKS // 2, body, 0)

    plsc.subcore_barrier()
    pltpu.sync_copy(acc_sh.at[pl.ds(sid * RPT, RPT)],
                    out_hbm.at[pl.ds(cid * NP + sid * RPT, RPT)])


_conv_kernel = pl.kernel(_conv_body, **_CONV_KERNEL_ARGS)


def _tc1a_body(x_ref, fcwt_ref, fcb_ref, w1t_ref, t1_ref):
    h0 = jnp.dot(x_ref[...], fcwt_ref[...],
                 preferred_element_type=jnp.float32) + fcb_ref[...]
    t1_ref[...] = jnp.dot(h0, w1t_ref[...], preferred_element_type=jnp.float32)


def _tc1b_body(t1_ref, degp_ref, u1_ref, dis_ref):
    deg = degp_ref[0, 0:N, 0:1] + degp_ref[1, 0:N, 0:1] + 1.0
    dis = lax.rsqrt(deg)
    dis_ref[...] = dis
    u1_ref[...] = t1_ref[...] * dis


def _tc2_body(acc_ref, u_ref, dis_ref, b_ref, wt_ref, uo_ref):
    s = acc_ref[0, 0:N, :] + acc_ref[1, 0:N, :] + u_ref[...]
    dis = dis_ref[...]
    h = jnp.maximum(s * dis + b_ref[...], 0.0)
    t = jnp.dot(h, wt_ref[...], preferred_element_type=jnp.float32)
    uo_ref[...] = t * dis


def _tc3_body(acc_ref, u_ref, dis_ref, b_ref, batch_ref, out_ref):
    s = acc_ref[0, 0:N, :] + acc_ref[1, 0:N, :] + u_ref[...]
    h = jnp.maximum(s * dis_ref[...] + b_ref[...], 0.0)
    seg = lax.broadcasted_iota(jnp.int32, (G, N), 0)
    onehot = (seg == jnp.broadcast_to(batch_ref[...], (G, N))
              ).astype(jnp.float32)
    sums = jnp.dot(onehot, h, preferred_element_type=jnp.float32)
    counts = jnp.sum(onehot, axis=1, keepdims=True)
    out_ref[...] = sums / jnp.maximum(counts, 1.0)


_tc1a = pl.pallas_call(
    _tc1a_body, out_shape=jax.ShapeDtypeStruct((N, D), jnp.float32))

_tc1b = pl.pallas_call(
    _tc1b_body,
    out_shape=(jax.ShapeDtypeStruct((N, D), jnp.float32),
               jax.ShapeDtypeStruct((N, 1), jnp.float32)))

_tc2 = pl.pallas_call(
    _tc2_body, out_shape=jax.ShapeDtypeStruct((N, D), jnp.float32))

_tc3 = pl.pallas_call(
    _tc3_body, out_shape=jax.ShapeDtypeStruct((G, D), jnp.float32))


def kernel(x, edge_index, batch, fc_W, fc_b, W1, b1, W2, b2):
    pad = E_PAD - E
    row1 = jnp.concatenate(
        [edge_index[0], (jnp.arange(pad, dtype=jnp.int32) % N)])
    col3 = jnp.concatenate(
        [edge_index[1],
         N + (jnp.arange(pad, dtype=jnp.int32) % (NP - N))]).reshape(NW, CHUNKS, K)

    degp = _deg_kernel(col3).reshape(NC, NP, DEGW)
    t1 = _tc1a(x, fc_W.T, fc_b.reshape(1, D), W1.T)
    u1, dis = _tc1b(t1, degp)

    acc1 = _conv_kernel(u1, row1, col3).reshape(NC, NP, D)
    u2 = _tc2(acc1, u1, dis, b1.reshape(1, D), W2.T)

    acc2 = _conv_kernel(u2, row1, col3).reshape(NC, NP, D)
    out = _tc3(acc2, u2, dis, b2.reshape(1, D), batch.reshape(1, N))
    return out


# speedup vs baseline: 3.6544x; 1.1599x over previous
import functools

import jax
import jax.numpy as jnp
from jax import lax
from jax.experimental import pallas as pl
from jax.experimental.pallas import tpu as pltpu
from jax.experimental.pallas import tpu_sc as plsc

N = 10000
E = 320000
D = 128
G = 64

NC = 2
NS = 16
NW = NC * NS
EPW = 10240
E_PAD = NW * EPW
K = 128
CHUNKS = EPW // K
NP = 10240
RPT = NP // NS
ZR = 32
DEGW = 128

_mesh = plsc.VectorSubcoreMesh(
    core_axis_name="c", subcore_axis_name="s", num_cores=NC, num_subcores=NS)


def _fill_const(ref, rows, width, val):
    def body(i, _):
        r = i // (width // 16)
        c = (i % (width // 16)) * 16
        ref[r, pl.ds(c, 16)] = jnp.full((16,), val, jnp.float32)
        return 0
    lax.fori_loop(0, rows * (width // 16), body, 0)


_DEG_KERNEL_ARGS = dict(
    out_type=jax.ShapeDtypeStruct((NC * NP,), jnp.float32),
    mesh=_mesh,
    compiler_params=pltpu.CompilerParams(needs_layout_passes=False),
    scratch_types=[
        pltpu.VMEM((CHUNKS, K), jnp.int32),
        pltpu.VMEM((NP,), jnp.float32),
        pltpu.VMEM((NS, RPT), jnp.float32),
        pltpu.VMEM((RPT,), jnp.float32),
        pltpu.VMEM_SHARED((NS, NP), jnp.float32),
        pltpu.SemaphoreType.DMA,
    ],
)


def _deg_body(col3_hbm, out_hbm, colb, hist, redbuf, sums, shr, sem):
    cid = lax.axis_index("c")
    sid = lax.axis_index("s")
    wid = cid * NS + sid

    idx_cp = pltpu.async_copy(col3_hbm.at[wid], colb, sem)

    def zero(i, _):
        hist[pl.ds(i * 16, 16)] = jnp.zeros((16,), jnp.float32)
        return 0
    lax.fori_loop(0, NP // 16, zero, 0)
    idx_cp.wait()

    ones16 = jnp.ones((16,), jnp.float32)
    GPC = K // 16

    def scat(it, _):
        g = it // GPC
        j = it % GPC
        idx = colb[g, pl.ds(j * 16, 16)]
        plsc.addupdate_scatter(hist, [idx], ones16)
        return 0
    lax.fori_loop(0, CHUNKS * GPC, scat, 0)

    pltpu.sync_copy(hist, shr.at[sid])
    plsc.subcore_barrier()

    for r in range(NS):
        pltpu.sync_copy(shr.at[r, pl.ds(sid * RPT, RPT)], redbuf.at[r])

    def red(i, _):
        acc = jnp.zeros((16,), jnp.float32)
        for r in range(NS):
            acc = acc + redbuf[r, pl.ds(i * 16, 16)]
        sums[pl.ds(i * 16, 16)] = acc
        return 0
    lax.fori_loop(0, RPT // 16, red, 0)

    pltpu.sync_copy(sums, out_hbm.at[pl.ds(cid * NP + sid * RPT, RPT)])


_deg_kernel = pl.kernel(_deg_body, **_DEG_KERNEL_ARGS)


_CONV_KERNEL_ARGS = dict(
    out_type=jax.ShapeDtypeStruct((NC * NP, D), jnp.float32),
    mesh=_mesh,
    scratch_types=[
        pltpu.VMEM((K,), jnp.int32),
        pltpu.VMEM((K,), jnp.int32),
        pltpu.VMEM((CHUNKS, K), jnp.int32),
        pltpu.VMEM((K, D), jnp.float32),
        pltpu.VMEM((K, D), jnp.float32),
        pltpu.VMEM((ZR, D), jnp.float32),
        pltpu.VMEM_SHARED((NP, D), jnp.float32),
        pltpu.SemaphoreType.DMA,
        pltpu.SemaphoreType.DMA,
        pltpu.SemaphoreType.DMA,
        pltpu.SemaphoreType.DMA,
    ],
)


def _conv_body(u_hbm, row_hbm, col3_hbm, out_hbm,
               ri0, ri1, colb, r0_v, r1_v, zbuf_v, acc_sh,
               si0, si1, sg0, sg1):
    cid = lax.axis_index("c")
    sid = lax.axis_index("s")
    wid = cid * NS + sid
    ebase = wid * EPW

    col_cp = pltpu.async_copy(col3_hbm.at[wid], colb, sg1)
    _fill_const(zbuf_v, ZR, D, 0.0)
    for z in range(RPT // ZR):
        pltpu.sync_copy(zbuf_v, acc_sh.at[pl.ds(sid * RPT + z * ZR, ZR)])
    col_cp.wait()
    plsc.subcore_barrier()

    def idx_load(g, buf, sem):
        return pltpu.async_copy(row_hbm.at[pl.ds(ebase + g * K, K)], buf, sem)

    def idx_wait(buf, sem):
        pltpu.make_async_copy(row_hbm.at[pl.ds(0, K)], buf, sem).wait()

    def gather(ibuf, buf, sem):
        return pltpu.async_copy(u_hbm.at[ibuf], buf, sem)

    def gwait(buf, sem):
        pltpu.make_async_copy(u_hbm.at[ri0], buf, sem).wait()

    def scatter(g, buf):
        pltpu.sync_copy(buf, acc_sh.at[colb.at[g]], add=True)

    idx_load(0, ri0, si0)
    idx_wait(ri0, si0)
    gather(ri0, r0_v, sg0)
    idx_load(1, ri1, si1)

    def body(i, _):
        a = 2 * i
        idx_wait(ri1, si1)
        gather(ri1, r1_v, sg1)
        gwait(r0_v, sg0)

        @pl.when(a + 2 < CHUNKS)
        def _():
            idx_load(a + 2, ri0, si0)
        scatter(a, r0_v)

        @pl.when(a + 2 < CHUNKS)
        def _():
            idx_wait(ri0, si0)
            gather(ri0, r0_v, sg0)
        gwait(r1_v, sg1)

        @pl.when(a + 3 < CHUNKS)
        def _():
            idx_load(a + 3, ri1, si1)
        scatter(a + 1, r1_v)
        return 0
    lax.fori_loop(0, CHUNKS // 2, body, 0)

    plsc.subcore_barrier()
    pltpu.sync_copy(acc_sh.at[pl.ds(sid * RPT, RPT)],
                    out_hbm.at[pl.ds(cid * NP + sid * RPT, RPT)])


_conv_kernel = pl.kernel(_conv_body, **_CONV_KERNEL_ARGS)


def _tc1a_body(x_ref, fcwt_ref, fcb_ref, w1t_ref, t1_ref):
    h0 = jnp.dot(x_ref[...], fcwt_ref[...],
                 preferred_element_type=jnp.float32) + fcb_ref[...]
    t1_ref[...] = jnp.dot(h0, w1t_ref[...], preferred_element_type=jnp.float32)


def _tc1b_body(t1_ref, degp_ref, u1_ref, dis_ref):
    deg = degp_ref[0, 0:N, :] + degp_ref[1, 0:N, :] + 1.0
    dis = lax.rsqrt(deg)
    dis_ref[...] = dis
    u1_ref[...] = t1_ref[...] * dis


def _tc2_body(acc_ref, u_ref, dis_ref, b_ref, wt_ref, uo_ref):
    s = acc_ref[0, 0:N, :] + acc_ref[1, 0:N, :] + u_ref[...]
    dis = dis_ref[...]
    h = jnp.maximum(s * dis + b_ref[...], 0.0)
    t = jnp.dot(h, wt_ref[...], preferred_element_type=jnp.float32)
    uo_ref[...] = t * dis


def _tc3_body(acc_ref, u_ref, dis_ref, b_ref, batch_ref, out_ref):
    s = acc_ref[0, 0:N, :] + acc_ref[1, 0:N, :] + u_ref[...]
    h = jnp.maximum(s * dis_ref[...] + b_ref[...], 0.0)
    seg = lax.broadcasted_iota(jnp.int32, (G, N), 0)
    onehot = (seg == jnp.broadcast_to(batch_ref[...], (G, N))
              ).astype(jnp.float32)
    sums = jnp.dot(onehot, h, preferred_element_type=jnp.float32)
    counts = jnp.sum(onehot, axis=1, keepdims=True)
    out_ref[...] = sums / jnp.maximum(counts, 1.0)


_tc1a = pl.pallas_call(
    _tc1a_body, out_shape=jax.ShapeDtypeStruct((N, D), jnp.float32))

_tc1b = pl.pallas_call(
    _tc1b_body,
    out_shape=(jax.ShapeDtypeStruct((N, D), jnp.float32),
               jax.ShapeDtypeStruct((N, 1), jnp.float32)))

_tc2 = pl.pallas_call(
    _tc2_body, out_shape=jax.ShapeDtypeStruct((N, D), jnp.float32))

_tc3 = pl.pallas_call(
    _tc3_body, out_shape=jax.ShapeDtypeStruct((G, D), jnp.float32))


def kernel(x, edge_index, batch, fc_W, fc_b, W1, b1, W2, b2):
    pad = E_PAD - E
    row1 = jnp.concatenate(
        [edge_index[0], (jnp.arange(pad, dtype=jnp.int32) % N)])
    col3 = jnp.concatenate(
        [edge_index[1],
         N + (jnp.arange(pad, dtype=jnp.int32) % (NP - N))]).reshape(NW, CHUNKS, K)

    degp = _deg_kernel(col3).reshape(NC, NP, 1)
    t1 = _tc1a(x, fc_W.T, fc_b.reshape(1, D), W1.T)
    u1, dis = _tc1b(t1, degp)

    acc1 = _conv_kernel(u1, row1, col3).reshape(NC, NP, D)
    u2 = _tc2(acc1, u1, dis, b1.reshape(1, D), W2.T)

    acc2 = _conv_kernel(u2, row1, col3).reshape(NC, NP, D)
    out = _tc3(acc2, u2, dis, b2.reshape(1, D), batch.reshape(1, N))
    return out


# speedup vs baseline: 3.6633x; 1.0025x over previous
import functools

import jax
import jax.numpy as jnp
from jax import lax
from jax.experimental import pallas as pl
from jax.experimental.pallas import tpu as pltpu
from jax.experimental.pallas import tpu_sc as plsc

N = 10000
E = 320000
D = 128
G = 64

NC = 2
NS = 16
NW = NC * NS
EPW = 10240
E_PAD = NW * EPW
K = 128
CHUNKS = EPW // K
NP = 10240
RPT = NP // NS
ZR = 32
DEGW = 128

_mesh = plsc.VectorSubcoreMesh(
    core_axis_name="c", subcore_axis_name="s", num_cores=NC, num_subcores=NS)


def _fill_const(ref, rows, width, val):
    def body(i, _):
        r = i // (width // 16)
        c = (i % (width // 16)) * 16
        ref[r, pl.ds(c, 16)] = jnp.full((16,), val, jnp.float32)
        return 0
    lax.fori_loop(0, rows * (width // 16), body, 0)


_DEG_KERNEL_ARGS = dict(
    out_type=jax.ShapeDtypeStruct((NC * NP,), jnp.float32),
    mesh=_mesh,
    compiler_params=pltpu.CompilerParams(needs_layout_passes=False),
    scratch_types=[
        pltpu.VMEM((CHUNKS, K), jnp.int32),
        pltpu.VMEM((NP,), jnp.float32),
        pltpu.VMEM((NS, RPT), jnp.float32),
        pltpu.VMEM((RPT,), jnp.float32),
        pltpu.VMEM_SHARED((NS, NP), jnp.float32),
        pltpu.SemaphoreType.DMA,
    ],
)


def _deg_body(col3_hbm, out_hbm, colb, hist, redbuf, sums, shr, sem):
    cid = lax.axis_index("c")
    sid = lax.axis_index("s")
    wid = cid * NS + sid

    idx_cp = pltpu.async_copy(col3_hbm.at[wid], colb, sem)

    def zero(i, _):
        hist[pl.ds(i * 16, 16)] = jnp.zeros((16,), jnp.float32)
        return 0
    lax.fori_loop(0, NP // 16, zero, 0)
    idx_cp.wait()

    ones16 = jnp.ones((16,), jnp.float32)
    GPC = K // 16

    def scat(it, _):
        g = it // GPC
        j = it % GPC
        idx = colb[g, pl.ds(j * 16, 16)]
        plsc.addupdate_scatter(hist, [idx], ones16)
        return 0
    lax.fori_loop(0, CHUNKS * GPC, scat, 0)

    pltpu.sync_copy(hist, shr.at[sid])
    plsc.subcore_barrier()

    for r in range(NS):
        pltpu.sync_copy(shr.at[r, pl.ds(sid * RPT, RPT)], redbuf.at[r])

    def red(i, _):
        acc = jnp.zeros((16,), jnp.float32)
        for r in range(NS):
            acc = acc + redbuf[r, pl.ds(i * 16, 16)]
        sums[pl.ds(i * 16, 16)] = acc
        return 0
    lax.fori_loop(0, RPT // 16, red, 0)

    pltpu.sync_copy(sums, out_hbm.at[pl.ds(cid * NP + sid * RPT, RPT)])


_deg_kernel = pl.kernel(_deg_body, **_DEG_KERNEL_ARGS)


_CONV_KERNEL_ARGS = dict(
    out_type=jax.ShapeDtypeStruct((NC * NP, D), jnp.float32),
    mesh=_mesh,
    scratch_types=[
        pltpu.VMEM((K,), jnp.int32),
        pltpu.VMEM((K,), jnp.int32),
        pltpu.VMEM((CHUNKS, K), jnp.int32),
        pltpu.VMEM((K, D), jnp.float32),
        pltpu.VMEM((K, D), jnp.float32),
        pltpu.VMEM((ZR, D), jnp.float32),
        pltpu.VMEM_SHARED((NP, D), jnp.float32),
        pltpu.SemaphoreType.DMA,
        pltpu.SemaphoreType.DMA,
        pltpu.SemaphoreType.DMA,
        pltpu.SemaphoreType.DMA,
    ],
)


def _conv_body(u_hbm, row_hbm, col3_hbm, out_hbm,
               ri0, ri1, colb, r0_v, r1_v, zbuf_v, acc_sh,
               si0, si1, sg0, sg1):
    cid = lax.axis_index("c")
    sid = lax.axis_index("s")
    wid = cid * NS + sid
    ebase = wid * EPW

    col_cp = pltpu.async_copy(col3_hbm.at[wid], colb, sg1)
    _fill_const(zbuf_v, ZR, D, 0.0)
    for z in range(RPT // ZR):
        pltpu.sync_copy(zbuf_v, acc_sh.at[pl.ds(sid * RPT + z * ZR, ZR)])
    col_cp.wait()
    plsc.subcore_barrier()

    def idx_load(g, buf, sem):
        return pltpu.async_copy(row_hbm.at[pl.ds(ebase + g * K, K)], buf, sem)

    def idx_wait(buf, sem):
        pltpu.make_async_copy(row_hbm.at[pl.ds(0, K)], buf, sem).wait()

    def gather(ibuf, buf, sem):
        return pltpu.async_copy(u_hbm.at[ibuf], buf, sem)

    def gwait(buf, sem):
        pltpu.make_async_copy(u_hbm.at[ri0], buf, sem).wait()

    def scatter(g, buf):
        pltpu.sync_copy(buf, acc_sh.at[colb.at[g]], add=True)

    idx_load(0, ri0, si0)
    idx_wait(ri0, si0)
    gather(ri0, r0_v, sg0)
    idx_load(1, ri1, si1)

    def body(i, _):
        a = 2 * i
        idx_wait(ri1, si1)
        gather(ri1, r1_v, sg1)
        gwait(r0_v, sg0)

        @pl.when(a + 2 < CHUNKS)
        def _():
            idx_load(a + 2, ri0, si0)
        scatter(a, r0_v)

        @pl.when(a + 2 < CHUNKS)
        def _():
            idx_wait(ri0, si0)
            gather(ri0, r0_v, sg0)
        gwait(r1_v, sg1)

        @pl.when(a + 3 < CHUNKS)
        def _():
            idx_load(a + 3, ri1, si1)
        scatter(a + 1, r1_v)
        return 0
    lax.fori_loop(0, CHUNKS // 2, body, 0)

    plsc.subcore_barrier()
    pltpu.sync_copy(acc_sh.at[pl.ds(sid * RPT, RPT)],
                    out_hbm.at[pl.ds(cid * NP + sid * RPT, RPT)])


_conv_kernel = pl.kernel(_conv_body, **_CONV_KERNEL_ARGS)


def _tc1_body(x_ref, fcwt_ref, fcb_ref, w1t_ref, degp_ref, u1_ref, dis_ref):
    deg = degp_ref[0, 0:N, :] + degp_ref[1, 0:N, :] + 1.0
    dis = lax.rsqrt(deg)
    dis_ref[...] = dis
    h0 = jnp.dot(x_ref[...], fcwt_ref[...],
                 preferred_element_type=jnp.float32) + fcb_ref[...]
    t1 = jnp.dot(h0, w1t_ref[...], preferred_element_type=jnp.float32)
    u1_ref[...] = t1 * dis


def _tc2_body(acc_ref, u_ref, dis_ref, b_ref, wt_ref, uo_ref):
    s = acc_ref[0, 0:N, :] + acc_ref[1, 0:N, :] + u_ref[...]
    dis = dis_ref[...]
    h = jnp.maximum(s * dis + b_ref[...], 0.0)
    t = jnp.dot(h, wt_ref[...], preferred_element_type=jnp.float32)
    uo_ref[...] = t * dis


def _tc3_body(acc_ref, u_ref, dis_ref, b_ref, batch_ref, out_ref):
    s = acc_ref[0, 0:N, :] + acc_ref[1, 0:N, :] + u_ref[...]
    h = jnp.maximum(s * dis_ref[...] + b_ref[...], 0.0)
    seg = lax.broadcasted_iota(jnp.int32, (G, N), 0)
    onehot = (seg == jnp.broadcast_to(batch_ref[...], (G, N))
              ).astype(jnp.float32)
    sums = jnp.dot(onehot, h, preferred_element_type=jnp.float32)
    counts = jnp.sum(onehot, axis=1, keepdims=True)
    out_ref[...] = sums / jnp.maximum(counts, 1.0)


_tc1 = pl.pallas_call(
    _tc1_body,
    out_shape=(jax.ShapeDtypeStruct((N, D), jnp.float32),
               jax.ShapeDtypeStruct((N, 1), jnp.float32)))

_tc2 = pl.pallas_call(
    _tc2_body, out_shape=jax.ShapeDtypeStruct((N, D), jnp.float32))

_tc3 = pl.pallas_call(
    _tc3_body, out_shape=jax.ShapeDtypeStruct((G, D), jnp.float32))


def kernel(x, edge_index, batch, fc_W, fc_b, W1, b1, W2, b2):
    pad = E_PAD - E
    row1 = jnp.concatenate(
        [edge_index[0], (jnp.arange(pad, dtype=jnp.int32) % N)])
    col3 = jnp.concatenate(
        [edge_index[1],
         N + (jnp.arange(pad, dtype=jnp.int32) % (NP - N))]).reshape(NW, CHUNKS, K)

    degp = _deg_kernel(col3).reshape(NC, NP, 1)
    u1, dis = _tc1(x, fc_W.T, fc_b.reshape(1, D), W1.T, degp)

    acc1 = _conv_kernel(u1, row1, col3).reshape(NC, NP, D)
    u2 = _tc2(acc1, u1, dis, b1.reshape(1, D), W2.T)

    acc2 = _conv_kernel(u2, row1, col3).reshape(NC, NP, D)
    out = _tc3(acc2, u2, dis, b2.reshape(1, D), batch.reshape(1, N))
    return out
